# baseline (speedup 1.0000x reference)
import functools

import jax
import jax.numpy as jnp
from jax import lax
from jax.experimental import pallas as pl
from jax.experimental.pallas import tpu as pltpu
from jax.experimental.pallas import tpu_sc as plsc

T = 2048
H = 1024
E = 8
K = 2
I = 512
TK = T * K
BM = 256
BMG = 128
PT = 5120
NT = PT // BMG
NC = 2
NS = 16
NW = NC * NS
CHUNK = TK // NS
RPW = PT // NW
TPW = T // NW


def _router_body(x_ref, rwt_ref, ids_ref, wk_ref):
    x = x_ref[...]
    logits = lax.dot_general(
        x, rwt_ref[...], (((1,), (0,)), ((), ())),
        precision=lax.Precision.DEFAULT, preferred_element_type=jnp.float32)
    m = jnp.max(logits, axis=-1, keepdims=True)
    ex = jnp.exp(logits - m)
    probs = ex / jnp.sum(ex, axis=-1, keepdims=True)
    lane = lax.broadcasted_iota(jnp.int32, probs.shape, 1)
    p1 = jnp.max(probs, axis=-1, keepdims=True)
    i1 = jnp.min(jnp.where(probs == p1, lane, E), axis=-1, keepdims=True)
    rest = jnp.where(lane == i1, -jnp.inf, probs)
    p2 = jnp.max(rest, axis=-1, keepdims=True)
    i2 = jnp.min(jnp.where(rest == p2, lane, E), axis=-1, keepdims=True)
    denom = p1 + p2
    ids_ref[...] = jnp.concatenate([i1, i2], axis=1)
    wk_ref[...] = jnp.concatenate([p1 / denom, p2 / denom], axis=1)


def _router(x, rwt):
    return pl.pallas_call(
        _router_body,
        grid=(T // BM,),
        in_specs=[
            pl.BlockSpec((BM, H), lambda i: (i, 0)),
            pl.BlockSpec((H, E), lambda i: (0, 0)),
        ],
        out_specs=[
            pl.BlockSpec((BM, K), lambda i: (i, 0)),
            pl.BlockSpec((BM, K), lambda i: (i, 0)),
        ],
        out_shape=[
            jax.ShapeDtypeStruct((T, K), jnp.int32),
            jax.ShapeDtypeStruct((T, K), jnp.float32),
        ],
    )(x, rwt)


def _zeros16():
    return jnp.zeros((16,), jnp.int32)


def _ones16():
    return jnp.full((16,), 1, jnp.int32)


def _mask16(m):
    return jnp.where(m, _ones16(), _zeros16())


def _cumsum16(v, lane):
    incl = v
    for sh in (1, 2, 4, 8):
        g = incl[jnp.maximum(lane - sh, 0)]
        incl = incl + jnp.where(lane >= sh, g, _zeros16())
    return incl


def _sc_dispatch_body(ids_hbm, xb_hbm, pos_hbm, eof_hbm, xs_hbm,
                      ids_v, cnt_v, call_v, pos_lo, pos_hi, tok_lo, tok_hi,
                      eof_v, idx_a, idx_b, rows_v, counts_sp, tok_sp, sem):
    c = lax.axis_index("c")
    s = lax.axis_index("s")
    lane = lax.iota(jnp.int32, 16)

    pltpu.sync_copy(ids_hbm.at[pl.ds(s * CHUNK, CHUNK)], ids_v)
    cnt = _zeros16()
    for e in range(E):
        acc = _zeros16()
        for ch in range(CHUNK // 16):
            v = ids_v[pl.ds(ch * 16, 16)]
            acc = acc + _mask16(v == e)
        tot_e = _cumsum16(acc, lane)[lane * 0 + 15]
        cnt = cnt + jnp.where(lane == e, tot_e, _zeros16())
    cnt_v[...] = cnt
    pltpu.sync_copy(cnt_v, counts_sp.at[pl.ds(s * 16, 16)])
    plsc.subcore_barrier()

    pltpu.sync_copy(counts_sp, call_v)
    tot = _zeros16()
    pref = _zeros16()
    for w in range(NS):
        row = call_v[pl.ds(w * 16, 16)]
        tot = tot + row
        pref = pref + row * jnp.where(w < s, 1, 0)
    shift = BMG.bit_length() - 1
    pc = lax.shift_left(lax.shift_right_logical(tot + (BMG - 1), shift), shift)
    cinc = _cumsum16(pc, lane)
    bvec = (cinc - pc) + pref

    for ch in range(CHUNK // 16):
        v = ids_v[pl.ds(ch * 16, 16)]
        posc = _zeros16()
        for e in range(E):
            m = v == e
            incl = _cumsum16(_mask16(m), lane)
            be = bvec[lane * 0 + e]
            posc = jnp.where(m, be + incl - 1, posc)
            tote = incl[lane * 0 + 15]
            bvec = bvec + jnp.where(lane == e, tote, _zeros16())
        gp = s * CHUNK + ch * 16 + lane
        tokc = lax.shift_right_logical(gp, 1)
        dst = pos_lo if ch < 8 else pos_hi
        dstt = tok_lo if ch < 8 else tok_hi
        off = (ch % 8) * 16
        dst[pl.ds(off, 16)] = posc
        dstt[pl.ds(off, 16)] = tokc

    pltpu.sync_copy(tok_lo, tok_sp.at[pos_lo])
    pltpu.sync_copy(tok_hi, tok_sp.at[pos_hi])

    @pl.when(c == 0)
    def _():
        pltpu.sync_copy(pos_lo, pos_hbm.at[pl.ds(s * CHUNK, 128)])
        pltpu.sync_copy(pos_hi, pos_hbm.at[pl.ds(s * CHUNK + 128, 128)])

    @pl.when(c + s == 0)
    def _():
        for chk in range(3):
            u = lane + chk * 16
            acc = _zeros16()
            for e in range(E):
                ce = cinc[lane * 0 + e]
                acc = acc + jnp.where(u * BMG >= ce, _ones16(), _zeros16())
            eof_v[pl.ds(chk * 16, 16)] = jnp.minimum(acc, E - 1)
        pltpu.sync_copy(eof_v.at[pl.ds(0, NT)], eof_hbm)

    plsc.subcore_barrier()

    w = c * NS + s
    pltpu.sync_copy(tok_sp.at[pl.ds(w * RPW, 80)], idx_a)
    for k in range(5):
        idx_a[pl.ds(k * 16, 16)] = jnp.clip(idx_a[pl.ds(k * 16, 16)], 0, T - 1)
    pltpu.sync_copy(tok_sp.at[pl.ds(w * RPW + 80, 80)], idx_b)
    for k in range(5):
        idx_b[pl.ds(k * 16, 16)] = jnp.clip(idx_b[pl.ds(k * 16, 16)], 0, T - 1)
    pltpu.async_copy(xb_hbm.at[idx_a], rows_v, sem).wait()
    pltpu.sync_copy(rows_v, xs_hbm.at[pl.ds(w * RPW, 80)])
    pltpu.async_copy(xb_hbm.at[idx_b], rows_v, sem).wait()
    pltpu.sync_copy(rows_v, xs_hbm.at[pl.ds(w * RPW + 80, 80)])


def _sc_dispatch(ids_flat, xb):
    mesh = plsc.VectorSubcoreMesh(core_axis_name="c", subcore_axis_name="s")
    f = pl.kernel(
        _sc_dispatch_body,
        out_type=[
            jax.ShapeDtypeStruct((TK,), jnp.int32),
            jax.ShapeDtypeStruct((NT,), jnp.int32),
            jax.ShapeDtypeStruct((PT, H), jnp.float32),
        ],
        mesh=mesh,
        scratch_types=[
            pltpu.VMEM((CHUNK,), jnp.int32),
            pltpu.VMEM((16,), jnp.int32),
            pltpu.VMEM((NS * 16,), jnp.int32),
            pltpu.VMEM((128,), jnp.int32),
            pltpu.VMEM((128,), jnp.int32),
            pltpu.VMEM((128,), jnp.int32),
            pltpu.VMEM((128,), jnp.int32),
            pltpu.VMEM((48,), jnp.int32),
            pltpu.VMEM((80,), jnp.int32),
            pltpu.VMEM((80,), jnp.int32),
            pltpu.VMEM((80, H), jnp.float32),
            pltpu.VMEM_SHARED((NS * 16,), jnp.int32),
            pltpu.VMEM_SHARED((PT,), jnp.int32),
            pltpu.SemaphoreType.DMA,
        ],
    )
    return f(ids_flat, xb)


def _shared_body(x_ref, sgu_ref, sd_ref, o_ref):
    xb = x_ref[...].astype(jnp.bfloat16)
    gu = lax.dot_general(xb, sgu_ref[...], (((1,), (0,)), ((), ())),
                         preferred_element_type=jnp.float32)
    g = gu[:, :I]
    u = gu[:, I:]
    act = (g / (1.0 + jnp.exp(-g))) * u
    o_ref[...] = lax.dot_general(
        act.astype(jnp.bfloat16), sd_ref[...], (((1,), (0,)), ((), ())),
        preferred_element_type=jnp.float32)


def _shared(x, sgu, sd):
    return pl.pallas_call(
        _shared_body,
        grid=(T // BM,),
        in_specs=[
            pl.BlockSpec((BM, H), lambda i: (i, 0)),
            pl.BlockSpec((H, 2 * I), lambda i: (0, 0)),
            pl.BlockSpec((I, H), lambda i: (0, 0)),
        ],
        out_specs=pl.BlockSpec((BM, H), lambda i: (i, 0)),
        out_shape=jax.ShapeDtypeStruct((T, H), jnp.float32),
    )(x, sgu, sd)


def _grouped_body(eof_ref, xs_ref, wgu_ref, wd_ref, ys_ref):
    xb = xs_ref[...].astype(jnp.bfloat16)
    gu = lax.dot_general(xb, wgu_ref[0], (((1,), (0,)), ((), ())),
                         preferred_element_type=jnp.float32)
    g = gu[:, :I]
    u = gu[:, I:]
    act = (g / (1.0 + jnp.exp(-g))) * u
    ys_ref[...] = lax.dot_general(
        act.astype(jnp.bfloat16), wd_ref[0], (((1,), (0,)), ((), ())),
        preferred_element_type=jnp.float32)


def _grouped(eof, xs, wgu, wd):
    grid_spec = pltpu.PrefetchScalarGridSpec(
        num_scalar_prefetch=1,
        grid=(NT,),
        in_specs=[
            pl.BlockSpec((BMG, H), lambda i, eof: (i, 0)),
            pl.BlockSpec((1, H, 2 * I), lambda i, eof: (eof[i], 0, 0)),
            pl.BlockSpec((1, I, H), lambda i, eof: (eof[i], 0, 0)),
        ],
        out_specs=pl.BlockSpec((BMG, H), lambda i, eof: (i, 0)),
    )
    return pl.pallas_call(
        _grouped_body,
        grid_spec=grid_spec,
        out_shape=jax.ShapeDtypeStruct((PT, H), jnp.float32),
    )(eof, xs, wgu, wd)


def _sc_combine_body(ys_hbm, sh_hbm, pos_hbm, wk_hbm, out_hbm,
                     posw_v, wkw_v, idx32_v, ys_v, sh_v, out_v, sem):
    c = lax.axis_index("c")
    s = lax.axis_index("s")
    w = c * NS + s
    tb = w * TPW
    pltpu.sync_copy(pos_hbm.at[pl.ds(tb * K, TPW * K)], posw_v)
    pltpu.sync_copy(wk_hbm.at[pl.ds(tb * K, TPW * K)], wkw_v)
    for sb in range(TPW // 16):
        idx32_v[pl.ds(0, 16)] = posw_v[pl.ds(sb * 32, 16)]
        idx32_v[pl.ds(16, 16)] = posw_v[pl.ds(sb * 32 + 16, 16)]
        pltpu.async_copy(ys_hbm.at[idx32_v], ys_v, sem).wait()
        pltpu.sync_copy(sh_hbm.at[pl.ds(tb + sb * 16, 16)], sh_v)
        wk_c0 = wkw_v[pl.ds(sb * 32, 16)]
        wk_c1 = wkw_v[pl.ds(sb * 32 + 16, 16)]
        for t in range(16):
            wch = wk_c0 if 2 * t < 16 else wk_c1
            w0 = wch[(2 * t) % 16]
            w1 = wch[(2 * t + 1) % 16]

            def hbody(hi, _, t=t, w0=w0, w1=w1):
                hs = pl.ds(hi * 16, 16)
                o = (sh_v[t, hs] + w0 * ys_v[2 * t, hs]
                     + w1 * ys_v[2 * t + 1, hs])
                out_v[t, hs] = o
                return 0

            lax.fori_loop(0, H // 16, hbody, 0)
        pltpu.sync_copy(out_v, out_hbm.at[pl.ds(tb + sb * 16, 16)])


def _sc_combine(ys, shared_out, pos, wk_flat):
    mesh = plsc.VectorSubcoreMesh(core_axis_name="c", subcore_axis_name="s")
    f = pl.kernel(
        _sc_combine_body,
        out_type=jax.ShapeDtypeStruct((T, H), jnp.float32),
        mesh=mesh,
        scratch_types=[
            pltpu.VMEM((TPW * K,), jnp.int32),
            pltpu.VMEM((TPW * K,), jnp.float32),
            pltpu.VMEM((32,), jnp.int32),
            pltpu.VMEM((32, H), jnp.float32),
            pltpu.VMEM((16, H), jnp.float32),
            pltpu.VMEM((16, H), jnp.float32),
            pltpu.SemaphoreType.DMA,
        ],
    )
    return f(ys, shared_out, pos, wk_flat)


@jax.jit
def kernel(hidden_states, router_weight, w_gate_up, w_down, shared_gate_up,
           shared_down):
    x = hidden_states
    rwt = router_weight.T.astype(jnp.float32)
    wgu = w_gate_up.astype(jnp.bfloat16)
    wd = w_down.astype(jnp.bfloat16)
    sgu = shared_gate_up.astype(jnp.bfloat16)
    sd = shared_down.astype(jnp.bfloat16)

    ids, wk = _router(x, rwt)
    pos, eof, xs = _sc_dispatch(ids.reshape(TK), x)
    shared_out = _shared(x, sgu, sd)
    ys = _grouped(eof, xs, wgu, wd)
    out = _sc_combine(ys, shared_out, pos, wk.reshape(TK))
    return out


# speedup vs baseline: 1.8280x; 1.8280x over previous
import functools

import jax
import jax.numpy as jnp
from jax import lax
from jax.experimental import pallas as pl
from jax.experimental.pallas import tpu as pltpu

T = 2048
H = 1024
E = 8
K = 2
I = 512
BM = 256
NSL = E + 1


def _moe_body(x_ref, rwt_ref, wgu_ref, sgu_ref, wdall_ref, o_ref, act_ref):
    x = x_ref[...]

    logits = lax.dot_general(
        x, rwt_ref[...], (((1,), (0,)), ((), ())),
        precision=lax.Precision.DEFAULT,
        preferred_element_type=jnp.float32,
    )
    m = jnp.max(logits, axis=-1, keepdims=True)
    ex = jnp.exp(logits - m)
    probs = ex / jnp.sum(ex, axis=-1, keepdims=True)

    lane = lax.broadcasted_iota(jnp.int32, probs.shape, 1)
    p1 = jnp.max(probs, axis=-1, keepdims=True)
    i1 = jnp.min(jnp.where(probs == p1, lane, E), axis=-1, keepdims=True)
    mask1 = lane == i1
    probs_rest = jnp.where(mask1, -jnp.inf, probs)
    p2 = jnp.max(probs_rest, axis=-1, keepdims=True)
    i2 = jnp.min(jnp.where(probs_rest == p2, lane, E), axis=-1, keepdims=True)
    mask2 = lane == i2
    denom = p1 + p2
    combine = (jnp.where(mask1, p1, 0.0) + jnp.where(mask2, p2, 0.0)) / denom

    xb = x.astype(jnp.bfloat16)

    def act_slice(gu, scale):
        g = gu[:, :I]
        u = gu[:, I:]
        a = (0.5 * g) * (1.0 + jnp.tanh(0.5 * g)) * u
        if scale is not None:
            a = a * scale
        return a.astype(jnp.bfloat16)

    for e in range(E):
        gu = lax.dot_general(
            xb, wgu_ref[e], (((1,), (0,)), ((), ())),
            preferred_element_type=jnp.float32)
        act_ref[:, e * I:(e + 1) * I] = act_slice(gu, combine[:, e:e + 1])
    gu = lax.dot_general(
        xb, sgu_ref[...], (((1,), (0,)), ((), ())),
        preferred_element_type=jnp.float32)
    act_ref[:, E * I:(E + 1) * I] = act_slice(gu, None)

    o_ref[...] = lax.dot_general(
        act_ref[...], wdall_ref[...], (((1,), (0,)), ((), ())),
        preferred_element_type=jnp.float32)


@jax.jit
def kernel(hidden_states, router_weight, w_gate_up, w_down, shared_gate_up,
           shared_down):
    rwt = router_weight.T.astype(jnp.float32)
    wgu = w_gate_up.astype(jnp.bfloat16)
    sgu = shared_gate_up.astype(jnp.bfloat16)
    wdall = jnp.concatenate(
        [w_down.reshape(E * I, H), shared_down], axis=0).astype(jnp.bfloat16)

    grid = (T // BM,)
    out = pl.pallas_call(
        _moe_body,
        grid=grid,
        in_specs=[
            pl.BlockSpec((BM, H), lambda i: (i, 0)),
            pl.BlockSpec((H, E), lambda i: (0, 0)),
            pl.BlockSpec((E, H, 2 * I), lambda i: (0, 0, 0)),
            pl.BlockSpec((H, 2 * I), lambda i: (0, 0)),
            pl.BlockSpec((NSL * I, H), lambda i: (0, 0)),
        ],
        out_specs=pl.BlockSpec((BM, H), lambda i: (i, 0)),
        out_shape=jax.ShapeDtypeStruct((T, H), jnp.float32),
        scratch_shapes=[pltpu.VMEM((BM, NSL * I), jnp.bfloat16)],
    )(hidden_states, rwt, wgu, sgu, wdall)
    return out
